# baseline (speedup 1.0000x reference)
import functools

import jax
import jax.numpy as jnp
from jax import lax
from jax.experimental import pallas as pl
from jax.experimental.pallas import tpu as pltpu
from jax.experimental.pallas import tpu_sc as plsc

_NC = 2
_NS = 16
_NW = _NC * _NS
_L = 16


def _sc_loss(x_hbm, y_hbm, o_hbm, y_v, buf, loss_v, sem0, sem1, *, B, C):
    b_per_w = B // _NW
    n_chunks = b_per_w // _L
    wid = lax.axis_index("s") * _NC + lax.axis_index("c")
    base = wid * b_per_w
    lanes = lax.iota(jnp.int32, _L)
    n_full = C // _L
    rem = C % _L
    sems = (sem0, sem1)

    pltpu.sync_copy(y_hbm.at[pl.ds(base, b_per_w)], y_v)

    def start(g, b):
        pltpu.async_copy(x_hbm.at[pl.ds(base + g * _L, _L), :], buf.at[b], sems[b])

    start(0, 0)
    start(1, 1)

    def do_pair(p, _):
        for b in (0, 1):
            g = 2 * p + b
            pltpu.make_async_copy(
                x_hbm.at[pl.ds(0, _L), :], buf.at[b], sems[b]).wait()
            y16 = y_v[pl.ds(g * _L, _L)]

            def row(i, lvec):
                y_s = jnp.sum(jnp.where(lanes == i, y16, 0))
                ybc = jnp.full((_L,), y_s)
                cb = (y_s // _L) * _L
                vdiag = buf[b, i, pl.ds(cb, _L)]
                oy = jnp.sum(jnp.where(lanes + cb == ybc, vdiag, 0.0))
                av = jnp.full((_L,), oy - 1.0)
                acc = jnp.zeros((_L,), jnp.float32)
                for t in range(n_full - 1 if rem else n_full):
                    v = buf[b, i, pl.ds(t * _L, _L)]
                    acc = acc + jnp.maximum(v - av, 0.0)
                if rem:
                    v = buf[b, i, pl.ds((n_full - 1) * _L, _L)]
                    acc = acc + jnp.maximum(v - av, 0.0)
                    v = buf[b, i, pl.ds(C - _L, _L)]
                    r = jnp.maximum(v - av, 0.0)
                    r = jnp.where(lanes >= (_L - rem), r, 0.0)
                    acc = acc + r
                rowsum = jnp.sum(acc)
                return jnp.where(lanes == i, (rowsum - 1.0) * (1.0 / C), lvec)

            lvec = lax.fori_loop(0, _L, row, jnp.zeros((_L,), jnp.float32))
            loss_v[pl.ds(g * _L, _L)] = lvec

            @pl.when(g + 2 < n_chunks)
            def _():
                pltpu.async_copy(
                    x_hbm.at[pl.ds(base + (g + 2) * _L, _L), :], buf.at[b], sems[b])

        return None

    lax.fori_loop(0, n_chunks // 2, do_pair, None)
    pltpu.sync_copy(loss_v, o_hbm.at[pl.ds(base, b_per_w)])


def kernel(output, y):
    B, C = output.shape
    b_per_w = B // _NW
    mesh = plsc.VectorSubcoreMesh(core_axis_name="c", subcore_axis_name="s")
    return pl.kernel(
        functools.partial(_sc_loss, B=B, C=C),
        out_type=jax.ShapeDtypeStruct((B,), jnp.float32),
        mesh=mesh,
        compiler_params=pltpu.CompilerParams(needs_layout_passes=False),
        scratch_types=[
            pltpu.VMEM((b_per_w,), jnp.int32),
            pltpu.VMEM((2, _L, C), jnp.float32),
            pltpu.VMEM((b_per_w,), jnp.float32),
            pltpu.SemaphoreType.DMA,
            pltpu.SemaphoreType.DMA,
        ],
    )(output, y)


# speedup vs baseline: 1.1084x; 1.1084x over previous
import functools

import jax
import jax.numpy as jnp
from jax import lax
from jax.experimental import pallas as pl
from jax.experimental.pallas import tpu as pltpu
from jax.experimental.pallas import tpu_sc as plsc

_NC = 2
_NS = 16
_NW = _NC * _NS
_L = 16

_SC_ROWS = 4096
_TC_BLOCK = 2048


def _tc_body(x_ref, y_ref, o_ref, *, C):
    x = x_ref[...]
    yv = y_ref[...]
    R = x.shape[0]
    col = jax.lax.broadcasted_iota(jnp.int32, (R, C), 1)
    onehot = col == yv[:, None]
    oy = jnp.sum(jnp.where(onehot, x, 0.0), axis=1, keepdims=True)
    hinge = jnp.maximum(x - oy + 1.0, 0.0)
    o_ref[...] = (jnp.sum(hinge, axis=1) - 1.0) * (1.0 / C)


def _sc_loss(x_hbm, y_hbm, o_hbm, y_v, buf, loss_v, sem0, sem1, *,
             C, row0, sc_rows):
    b_per_w = sc_rows // _NW
    n_chunks = b_per_w // _L
    wid = lax.axis_index("s") * _NC + lax.axis_index("c")
    base_o = wid * b_per_w
    base_x = row0 + base_o
    lanes = lax.iota(jnp.int32, _L)
    n_full = C // _L
    rem = C % _L
    sems = (sem0, sem1)

    pltpu.sync_copy(y_hbm.at[pl.ds(base_x, b_per_w)], y_v)

    def start(g, b):
        pltpu.async_copy(x_hbm.at[pl.ds(base_x + g * _L, _L), :], buf.at[b], sems[b])

    start(0, 0)
    start(1, 1)

    def do_pair(p, _):
        for b in (0, 1):
            g = 2 * p + b
            pltpu.make_async_copy(
                x_hbm.at[pl.ds(0, _L), :], buf.at[b], sems[b]).wait()
            y16 = y_v[pl.ds(g * _L, _L)]

            def row(i, lvec):
                y_s = jnp.sum(jnp.where(lanes == i, y16, 0))
                ybc = jnp.full((_L,), y_s)
                cb = (y_s // _L) * _L
                vdiag = buf[b, i, pl.ds(cb, _L)]
                oy = jnp.sum(jnp.where(lanes + cb == ybc, vdiag, 0.0))
                av = jnp.full((_L,), oy - 1.0)
                acc = jnp.zeros((_L,), jnp.float32)
                for t in range(n_full - 1 if rem else n_full):
                    v = buf[b, i, pl.ds(t * _L, _L)]
                    acc = acc + jnp.maximum(v - av, 0.0)
                if rem:
                    v = buf[b, i, pl.ds((n_full - 1) * _L, _L)]
                    acc = acc + jnp.maximum(v - av, 0.0)
                    v = buf[b, i, pl.ds(C - _L, _L)]
                    r = jnp.maximum(v - av, 0.0)
                    r = jnp.where(lanes >= (_L - rem), r, 0.0)
                    acc = acc + r
                rowsum = jnp.sum(acc)
                return jnp.where(lanes == i, (rowsum - 1.0) * (1.0 / C), lvec)

            lvec = lax.fori_loop(0, _L, row, jnp.zeros((_L,), jnp.float32))
            loss_v[pl.ds(g * _L, _L)] = lvec

            @pl.when(g + 2 < n_chunks)
            def _():
                pltpu.async_copy(
                    x_hbm.at[pl.ds(base_x + (g + 2) * _L, _L), :],
                    buf.at[b], sems[b])

        return None

    lax.fori_loop(0, n_chunks // 2, do_pair, None)
    pltpu.sync_copy(loss_v, o_hbm.at[pl.ds(base_o, b_per_w)])


def kernel(output, y):
    B, C = output.shape
    tc_rows = B - _SC_ROWS
    b_per_w = _SC_ROWS // _NW

    mesh = plsc.VectorSubcoreMesh(core_axis_name="c", subcore_axis_name="s")
    sc_part = pl.kernel(
        functools.partial(_sc_loss, C=C, row0=tc_rows, sc_rows=_SC_ROWS),
        out_type=jax.ShapeDtypeStruct((_SC_ROWS,), jnp.float32),
        mesh=mesh,
        compiler_params=pltpu.CompilerParams(needs_layout_passes=False),
        scratch_types=[
            pltpu.VMEM((b_per_w,), jnp.int32),
            pltpu.VMEM((2, _L, C), jnp.float32),
            pltpu.VMEM((b_per_w,), jnp.float32),
            pltpu.SemaphoreType.DMA,
            pltpu.SemaphoreType.DMA,
        ],
    )(output, y)

    tc_part = pl.pallas_call(
        functools.partial(_tc_body, C=C),
        grid=(tc_rows // _TC_BLOCK,),
        in_specs=[
            pl.BlockSpec((_TC_BLOCK, C), lambda i: (i, 0)),
            pl.BlockSpec((_TC_BLOCK,), lambda i: (i,)),
        ],
        out_specs=pl.BlockSpec((_TC_BLOCK,), lambda i: (i,)),
        out_shape=jax.ShapeDtypeStruct((tc_rows,), jnp.float32),
    )(output, y)

    return jnp.concatenate([tc_part, sc_part])


# speedup vs baseline: 1.1095x; 1.0010x over previous
import functools

import jax
import jax.numpy as jnp
from jax import lax
from jax.experimental import pallas as pl
from jax.experimental.pallas import tpu as pltpu
from jax.experimental.pallas import tpu_sc as plsc

_NC = 2
_NS = 16
_NW = _NC * _NS
_L = 16

_SC_ROWS = 4096
_TC_BLOCK = 2048


def _tc_body(x_ref, y_ref, o_ref, *, C):
    x = x_ref[...]
    yv = y_ref[...]
    R = x.shape[0]
    col = jax.lax.broadcasted_iota(jnp.int32, (R, C), 1)
    onehot = col == yv[:, None]
    oy = jnp.sum(jnp.where(onehot, x, 0.0), axis=1, keepdims=True)
    hinge = jnp.maximum(x - oy + 1.0, 0.0)
    o_ref[...] = (jnp.sum(hinge, axis=1) - 1.0) * (1.0 / C)


def _sc_loss(x_hbm, y_hbm, o_hbm, y_v, buf, loss_v, sem0, sem1, *,
             C, row0, sc_rows):
    b_per_w = sc_rows // _NW
    n_chunks = b_per_w // _L
    wid = lax.axis_index("s") * _NC + lax.axis_index("c")
    base_o = wid * b_per_w
    base_x = row0 + base_o
    lanes = lax.iota(jnp.int32, _L)
    n_full = C // _L
    rem = C % _L
    sems = (sem0, sem1)

    pltpu.sync_copy(y_hbm.at[pl.ds(base_x, b_per_w)], y_v)

    def start(g, b):
        pltpu.async_copy(x_hbm.at[pl.ds(base_x + g * _L, _L), :], buf.at[b], sems[b])

    start(0, 0)
    start(1, 1)

    def do_pair(p, _):
        for b in (0, 1):
            g = 2 * p + b
            pltpu.make_async_copy(
                x_hbm.at[pl.ds(0, _L), :], buf.at[b], sems[b]).wait()
            y16 = y_v[pl.ds(g * _L, _L)]

            def row(i, lvec):
                y_s = jnp.sum(jnp.where(lanes == i, y16, 0))
                ybc = jnp.full((_L,), y_s)
                cb = (y_s // _L) * _L
                vdiag = buf[b, i, pl.ds(cb, _L)]
                oy = jnp.sum(jnp.where(lanes + cb == ybc, vdiag, 0.0))
                av = jnp.full((_L,), oy - 1.0)
                acc = jnp.zeros((_L,), jnp.float32)
                for t in range(n_full - 1 if rem else n_full):
                    v = buf[b, i, pl.ds(t * _L, _L)]
                    acc = acc + jnp.maximum(v - av, 0.0)
                if rem:
                    v = buf[b, i, pl.ds((n_full - 1) * _L, _L)]
                    acc = acc + jnp.maximum(v - av, 0.0)
                    v = buf[b, i, pl.ds(C - _L, _L)]
                    r = jnp.maximum(v - av, 0.0)
                    r = jnp.where(lanes >= (_L - rem), r, 0.0)
                    acc = acc + r
                rowsum = jnp.sum(acc)
                return jnp.where(lanes == i, (rowsum - 1.0) * (1.0 / C), lvec)

            lvec = lax.fori_loop(0, _L, row, jnp.zeros((_L,), jnp.float32))
            loss_v[pl.ds(g * _L, _L)] = lvec

            @pl.when(g + 2 < n_chunks)
            def _():
                pltpu.async_copy(
                    x_hbm.at[pl.ds(base_x + (g + 2) * _L, _L), :],
                    buf.at[b], sems[b])

        return None

    lax.fori_loop(0, n_chunks // 2, do_pair, None)
    pltpu.sync_copy(loss_v, o_hbm.at[pl.ds(base_o, b_per_w)])


def kernel(output, y):
    B, C = output.shape
    tc_rows = B - _SC_ROWS
    b_per_w = _SC_ROWS // _NW

    mesh = plsc.VectorSubcoreMesh(core_axis_name="c", subcore_axis_name="s")
    sc_part = pl.kernel(
        functools.partial(_sc_loss, C=C, row0=tc_rows, sc_rows=_SC_ROWS),
        out_type=jax.ShapeDtypeStruct((_SC_ROWS,), jnp.float32),
        mesh=mesh,
        compiler_params=pltpu.CompilerParams(needs_layout_passes=False),
        cost_estimate=pl.CostEstimate(
            flops=2 * _SC_ROWS * C,
            transcendentals=0,
            bytes_accessed=4 * _SC_ROWS * C,
        ),
        scratch_types=[
            pltpu.VMEM((b_per_w,), jnp.int32),
            pltpu.VMEM((2, _L, C), jnp.float32),
            pltpu.VMEM((b_per_w,), jnp.float32),
            pltpu.SemaphoreType.DMA,
            pltpu.SemaphoreType.DMA,
        ],
    )(output, y)

    tc_part = pl.pallas_call(
        functools.partial(_tc_body, C=C),
        grid=(tc_rows // _TC_BLOCK,),
        in_specs=[
            pl.BlockSpec((_TC_BLOCK, C), lambda i: (i, 0)),
            pl.BlockSpec((_TC_BLOCK,), lambda i: (i,)),
        ],
        out_specs=pl.BlockSpec((_TC_BLOCK,), lambda i: (i,)),
        out_shape=jax.ShapeDtypeStruct((tc_rows,), jnp.float32),
    )(output, y)

    return jnp.concatenate([tc_part, sc_part])
